# baseline (speedup 1.0000x reference)
import jax
import jax.numpy as jnp
from jax import lax
from jax.experimental import pallas as pl
from jax.experimental.pallas import tpu as pltpu
from jax.experimental.pallas import tpu_sc as plsc

N, E, D, H, G = 10000, 320000, 128, 128, 64

NC, NS = 2, 16
NW = NC * NS
E_PER = E // NW
K = 50
CHUNKS = E_PER // K
N_PAD = 10240
ROWS_PER_TILE = N_PAD // NS
IDXB = 40
NGROUPS = CHUNKS // IDXB
DEPTH = 4
ZB = 40

_MESH = plsc.VectorSubcoreMesh(core_axis_name="c", subcore_axis_name="s")


def _make_sc_aggregate():
    def body(x_hbm, src_hbm, dst_hbm, zrow_hbm, aggr_out,
             sidx, didx, bufs, gsems, ssems, aggr_sh):
        c = lax.axis_index("c")
        s = lax.axis_index("s")
        wid = s * NC + c
        row0 = s * ROWS_PER_TILE
        stage = bufs.at[0, pl.ds(0, ZB)]

        pltpu.sync_copy(zrow_hbm, stage)

        @pl.loop(0, ROWS_PER_TILE // ZB)
        def _zero(i):
            pltpu.sync_copy(stage, aggr_sh.at[pl.ds(row0 + i * ZB, ZB)])

        plsc.subcore_barrier()

        @pl.loop(0, NGROUPS)
        def _group(g):
            pltpu.sync_copy(src_hbm.at[wid, g], sidx)
            pltpu.sync_copy(dst_hbm.at[wid, g], didx)

            @pl.loop(0, IDXB // DEPTH)
            def _round(p):
                j = p * DEPTH
                gathers = [
                    pltpu.async_copy(x_hbm.at[sidx.at[j + i]],
                                     bufs.at[i], gsems.at[i])
                    for i in range(DEPTH)
                ]
                scatters = []
                for i in range(DEPTH):
                    gathers[i].wait()
                    scatters.append(
                        pltpu.async_copy(bufs.at[i],
                                         aggr_sh.at[didx.at[j + i]],
                                         ssems.at[i], add=True))
                for cp in scatters:
                    cp.wait()

        plsc.subcore_barrier()

        @pl.loop(0, ROWS_PER_TILE // ZB)
        def _writeout(i):
            r0 = row0 + i * ZB
            pltpu.sync_copy(aggr_sh.at[pl.ds(r0, ZB)], stage)
            pltpu.sync_copy(stage, aggr_out.at[c, pl.ds(r0, ZB)])

    return pl.kernel(
        body,
        out_type=jax.ShapeDtypeStruct((NC, N_PAD, D), jnp.float32),
        mesh=_MESH,
        scratch_types=[pltpu.VMEM((IDXB, K), jnp.int32),
                       pltpu.VMEM((IDXB, K), jnp.int32),
                       pltpu.VMEM((DEPTH, K, D), jnp.float32),
                       pltpu.SemaphoreType.DMA((DEPTH,)),
                       pltpu.SemaphoreType.DMA((DEPTH,)),
                       pltpu.VMEM_SHARED((N_PAD, D), jnp.float32)])


def _make_sc_counts():
    def body(dst_hbm, zrow_hbm, ones_hbm, cnt_out,
             didx, ones_v, ssems, cnt_sh):
        c = lax.axis_index("c")
        s = lax.axis_index("s")
        wid = s * NC + c
        row0 = s * ROWS_PER_TILE
        stage = ones_v.at[pl.ds(0, ZB)]

        pltpu.sync_copy(zrow_hbm, stage)

        @pl.loop(0, ROWS_PER_TILE // ZB)
        def _zero(i):
            pltpu.sync_copy(stage, cnt_sh.at[pl.ds(row0 + i * ZB, ZB)])

        pltpu.sync_copy(ones_hbm, ones_v)
        plsc.subcore_barrier()

        @pl.loop(0, NGROUPS)
        def _group(g):
            pltpu.sync_copy(dst_hbm.at[wid, g], didx)

            @pl.loop(0, IDXB // DEPTH)
            def _round(p):
                j = p * DEPTH
                scatters = [
                    pltpu.async_copy(ones_v, cnt_sh.at[didx.at[j + i]],
                                     ssems.at[i], add=True)
                    for i in range(DEPTH)
                ]
                for cp in scatters:
                    cp.wait()

        plsc.subcore_barrier()

        @pl.loop(0, ROWS_PER_TILE // ZB)
        def _writeout(i):
            r0 = row0 + i * ZB
            pltpu.sync_copy(cnt_sh.at[pl.ds(r0, ZB)], stage)
            pltpu.sync_copy(stage, cnt_out.at[c, pl.ds(r0, ZB)])

    return pl.kernel(
        body,
        out_type=jax.ShapeDtypeStruct((NC, N_PAD, D), jnp.float32),
        mesh=_MESH,
        scratch_types=[pltpu.VMEM((IDXB, K), jnp.int32),
                       pltpu.VMEM((K, D), jnp.float32),
                       pltpu.SemaphoreType.DMA((DEPTH,)),
                       pltpu.VMEM_SHARED((N_PAD, D), jnp.float32)])


BLK = 1000


def _layer_body(x_ref, agg_ref, cnt_ref, wl_ref, wr_ref, bl_ref, o_ref):
    a = agg_ref[0] + agg_ref[1]
    cs = cnt_ref[0, :, 0:1] + cnt_ref[1, :, 0:1]
    rc = 1.0 / jnp.maximum(cs, 1.0)
    mean = a * rc
    h = (jnp.dot(mean, wl_ref[...], preferred_element_type=jnp.float32)
         + jnp.dot(x_ref[...], wr_ref[...], preferred_element_type=jnp.float32)
         + bl_ref[...])
    o_ref[...] = jnp.where(h >= 0, h, 0.1 * h)


_tc_layer = pl.pallas_call(
    _layer_body,
    grid=(N // BLK,),
    in_specs=[
        pl.BlockSpec((BLK, D), lambda i: (i, 0)),
        pl.BlockSpec((NC, BLK, D), lambda i: (0, i, 0)),
        pl.BlockSpec((NC, BLK, D), lambda i: (0, i, 0)),
        pl.BlockSpec((D, H), lambda i: (0, 0)),
        pl.BlockSpec((D, H), lambda i: (0, 0)),
        pl.BlockSpec((1, H), lambda i: (0, 0)),
    ],
    out_specs=pl.BlockSpec((BLK, H), lambda i: (i, 0)),
    out_shape=jax.ShapeDtypeStruct((N, H), jnp.float32),
)


def _pool_body(x_ref, b_ref, wo_ref, bo_ref, o_ref):
    bt = jnp.broadcast_to(b_ref[...], (G, N))
    gids = lax.broadcasted_iota(jnp.int32, (G, N), 0).astype(jnp.float32)
    mask = (bt == gids).astype(jnp.float32)
    pooled = jnp.dot(mask, x_ref[...], preferred_element_type=jnp.float32)
    cnt = jnp.sum(mask, axis=1, keepdims=True)
    pooled = pooled / jnp.maximum(cnt, 1.0)
    o_ref[...] = (jnp.dot(pooled, wo_ref[...],
                          preferred_element_type=jnp.float32) + bo_ref[...])


_tc_pool = pl.pallas_call(
    _pool_body,
    out_shape=jax.ShapeDtypeStruct((G, H), jnp.float32),
)


def kernel(node_features, edge_index, batch,
           W_l0, b_l0, W_r0, W_l1, b_l1, W_r1, W_l2, b_l2, W_r2,
           W_out, b_out):
    src = edge_index[0].reshape(NW, NGROUPS, IDXB, K)
    dst = edge_index[1].reshape(NW, NGROUPS, IDXB, K)
    zrow = jnp.zeros((ZB, D), jnp.float32)
    ones = jnp.ones((K, D), jnp.float32)

    sc_aggr = _make_sc_aggregate()
    sc_cnt = _make_sc_counts()

    x = node_features
    cnt = sc_cnt(dst, zrow, ones)
    aggr = sc_aggr(x, src, dst, zrow)
    x = _tc_layer(x, aggr, cnt, W_l0.T, W_r0.T, b_l0.reshape(1, H))
    aggr = sc_aggr(x, src, dst, zrow)
    x = _tc_layer(x, aggr, cnt, W_l1.T, W_r1.T, b_l1.reshape(1, H))
    aggr = sc_aggr(x, src, dst, zrow)
    x = _tc_layer(x, aggr, cnt, W_l2.T, W_r2.T, b_l2.reshape(1, H))

    wo = jnp.zeros((H, H), jnp.float32).at[:, 0].set(W_out[0])
    bo = jnp.broadcast_to(b_out[None, :], (1, H))
    out = _tc_pool(x, batch.astype(jnp.float32).reshape(1, N), wo, bo)
    return out[:, 0]


# speedup vs baseline: 1.0293x; 1.0293x over previous
import jax
import jax.numpy as jnp
from jax import lax
from jax.experimental import pallas as pl
from jax.experimental.pallas import tpu as pltpu
from jax.experimental.pallas import tpu_sc as plsc

N, E, D, H, G = 10000, 320000, 128, 128, 64

NC, NS = 2, 16
NW = NC * NS
E_PER = E // NW
K = 125
CHUNKS = E_PER // K
N_PAD = 10240
ROWS_PER_TILE = N_PAD // NS
IDXB = 16
NGROUPS = CHUNKS // IDXB
DEPTH = 2
ZB = 40

_MESH = plsc.VectorSubcoreMesh(core_axis_name="c", subcore_axis_name="s")


def _make_sc_aggregate():
    def body(x_hbm, src_hbm, dst_hbm, zrow_hbm, aggr_out,
             sidx, didx, bufs, gsems, ssems, aggr_sh):
        c = lax.axis_index("c")
        s = lax.axis_index("s")
        wid = s * NC + c
        row0 = s * ROWS_PER_TILE
        stage = bufs.at[0, pl.ds(0, ZB)]

        pltpu.sync_copy(zrow_hbm, stage)

        @pl.loop(0, ROWS_PER_TILE // ZB)
        def _zero(i):
            pltpu.sync_copy(stage, aggr_sh.at[pl.ds(row0 + i * ZB, ZB)])

        plsc.subcore_barrier()

        @pl.loop(0, NGROUPS)
        def _group(g):
            pltpu.sync_copy(src_hbm.at[wid, g], sidx)
            pltpu.sync_copy(dst_hbm.at[wid, g], didx)

            @pl.loop(0, IDXB // DEPTH)
            def _round(p):
                j = p * DEPTH
                gathers = [
                    pltpu.async_copy(x_hbm.at[sidx.at[j + i]],
                                     bufs.at[i], gsems.at[i])
                    for i in range(DEPTH)
                ]
                scatters = []
                for i in range(DEPTH):
                    gathers[i].wait()
                    scatters.append(
                        pltpu.async_copy(bufs.at[i],
                                         aggr_sh.at[didx.at[j + i]],
                                         ssems.at[i], add=True))
                for cp in scatters:
                    cp.wait()

        plsc.subcore_barrier()

        @pl.loop(0, ROWS_PER_TILE // ZB)
        def _writeout(i):
            r0 = row0 + i * ZB
            pltpu.sync_copy(aggr_sh.at[pl.ds(r0, ZB)], stage)
            pltpu.sync_copy(stage, aggr_out.at[c, pl.ds(r0, ZB)])

    return pl.kernel(
        body,
        out_type=jax.ShapeDtypeStruct((NC, N_PAD, D), jnp.float32),
        mesh=_MESH,
        scratch_types=[pltpu.VMEM((IDXB, K), jnp.int32),
                       pltpu.VMEM((IDXB, K), jnp.int32),
                       pltpu.VMEM((DEPTH, K, D), jnp.float32),
                       pltpu.SemaphoreType.DMA((DEPTH,)),
                       pltpu.SemaphoreType.DMA((DEPTH,)),
                       pltpu.VMEM_SHARED((N_PAD, D), jnp.float32)])


def _make_sc_counts():
    def body(dst_hbm, zrow_hbm, ones_hbm, cnt_out,
             didx, ones_v, ssems, cnt_sh):
        c = lax.axis_index("c")
        s = lax.axis_index("s")
        wid = s * NC + c
        row0 = s * ROWS_PER_TILE
        stage = ones_v.at[pl.ds(0, ZB)]

        pltpu.sync_copy(zrow_hbm, stage)

        @pl.loop(0, ROWS_PER_TILE // ZB)
        def _zero(i):
            pltpu.sync_copy(stage, cnt_sh.at[pl.ds(row0 + i * ZB, ZB)])

        pltpu.sync_copy(ones_hbm, ones_v)
        plsc.subcore_barrier()

        @pl.loop(0, NGROUPS)
        def _group(g):
            pltpu.sync_copy(dst_hbm.at[wid, g], didx)

            @pl.loop(0, IDXB // DEPTH)
            def _round(p):
                j = p * DEPTH
                scatters = [
                    pltpu.async_copy(ones_v, cnt_sh.at[didx.at[j + i]],
                                     ssems.at[i], add=True)
                    for i in range(DEPTH)
                ]
                for cp in scatters:
                    cp.wait()

        plsc.subcore_barrier()

        @pl.loop(0, ROWS_PER_TILE // ZB)
        def _writeout(i):
            r0 = row0 + i * ZB
            pltpu.sync_copy(cnt_sh.at[pl.ds(r0, ZB)], stage)
            pltpu.sync_copy(stage, cnt_out.at[c, pl.ds(r0, ZB)])

    return pl.kernel(
        body,
        out_type=jax.ShapeDtypeStruct((NC, N_PAD, D), jnp.float32),
        mesh=_MESH,
        scratch_types=[pltpu.VMEM((IDXB, K), jnp.int32),
                       pltpu.VMEM((K, D), jnp.float32),
                       pltpu.SemaphoreType.DMA((DEPTH,)),
                       pltpu.VMEM_SHARED((N_PAD, D), jnp.float32)])


BLK = 1000


def _layer_body(x_ref, agg_ref, cnt_ref, wl_ref, wr_ref, bl_ref, o_ref):
    a = agg_ref[0] + agg_ref[1]
    cs = cnt_ref[0, :, 0:1] + cnt_ref[1, :, 0:1]
    rc = 1.0 / jnp.maximum(cs, 1.0)
    mean = a * rc
    h = (jnp.dot(mean, wl_ref[...], preferred_element_type=jnp.float32)
         + jnp.dot(x_ref[...], wr_ref[...], preferred_element_type=jnp.float32)
         + bl_ref[...])
    o_ref[...] = jnp.where(h >= 0, h, 0.1 * h)


_tc_layer = pl.pallas_call(
    _layer_body,
    grid=(N // BLK,),
    in_specs=[
        pl.BlockSpec((BLK, D), lambda i: (i, 0)),
        pl.BlockSpec((NC, BLK, D), lambda i: (0, i, 0)),
        pl.BlockSpec((NC, BLK, D), lambda i: (0, i, 0)),
        pl.BlockSpec((D, H), lambda i: (0, 0)),
        pl.BlockSpec((D, H), lambda i: (0, 0)),
        pl.BlockSpec((1, H), lambda i: (0, 0)),
    ],
    out_specs=pl.BlockSpec((BLK, H), lambda i: (i, 0)),
    out_shape=jax.ShapeDtypeStruct((N, H), jnp.float32),
)


def _pool_body(x_ref, b_ref, wo_ref, bo_ref, o_ref):
    bt = jnp.broadcast_to(b_ref[...], (G, N))
    gids = lax.broadcasted_iota(jnp.int32, (G, N), 0).astype(jnp.float32)
    mask = (bt == gids).astype(jnp.float32)
    pooled = jnp.dot(mask, x_ref[...], preferred_element_type=jnp.float32)
    cnt = jnp.sum(mask, axis=1, keepdims=True)
    pooled = pooled / jnp.maximum(cnt, 1.0)
    o_ref[...] = (jnp.dot(pooled, wo_ref[...],
                          preferred_element_type=jnp.float32) + bo_ref[...])


_tc_pool = pl.pallas_call(
    _pool_body,
    out_shape=jax.ShapeDtypeStruct((G, H), jnp.float32),
)


def kernel(node_features, edge_index, batch,
           W_l0, b_l0, W_r0, W_l1, b_l1, W_r1, W_l2, b_l2, W_r2,
           W_out, b_out):
    src = edge_index[0].reshape(NW, NGROUPS, IDXB, K)
    dst = edge_index[1].reshape(NW, NGROUPS, IDXB, K)
    zrow = jnp.zeros((ZB, D), jnp.float32)
    ones = jnp.ones((K, D), jnp.float32)

    sc_aggr = _make_sc_aggregate()
    sc_cnt = _make_sc_counts()

    x = node_features
    cnt = sc_cnt(dst, zrow, ones)
    aggr = sc_aggr(x, src, dst, zrow)
    x = _tc_layer(x, aggr, cnt, W_l0.T, W_r0.T, b_l0.reshape(1, H))
    aggr = sc_aggr(x, src, dst, zrow)
    x = _tc_layer(x, aggr, cnt, W_l1.T, W_r1.T, b_l1.reshape(1, H))
    aggr = sc_aggr(x, src, dst, zrow)
    x = _tc_layer(x, aggr, cnt, W_l2.T, W_r2.T, b_l2.reshape(1, H))

    wo = jnp.zeros((H, H), jnp.float32).at[:, 0].set(W_out[0])
    bo = jnp.broadcast_to(b_out[None, :], (1, H))
    out = _tc_pool(x, batch.astype(jnp.float32).reshape(1, N), wo, bo)
    return out[:, 0]
